# Initial kernel scaffold; baseline (speedup 1.0000x reference)
import jax
import jax.numpy as jnp
from jax.experimental import pallas as pl


def kernel(x, edge_index, edge_type, nest_tensor, food_tensor, W_rel1, W_root1, b1, W_rel2, W_root2, b2, W_fc, b_fc, W_dir, b_dir, W_dist, b_dist):
    raise NotImplementedError("write your pallas kernel here")


# speedup vs baseline: 6.6670x; 6.6670x over previous
import functools

import jax
import jax.numpy as jnp
from jax import lax
from jax.experimental import pallas as pl
from jax.experimental.pallas import tpu as pltpu
from jax.experimental.pallas import tpu_sc as plsc

_F32 = jnp.float32
_L = 16
_CHUNK = 80
_ZCH = 80
_FCH = 80


def _mesh():
    return plsc.VectorSubcoreMesh(core_axis_name="c", subcore_axis_name="s")


@functools.lru_cache(maxsize=None)
def _make_edge_agg(n, e, r, emb, with_counts):
    info = plsc.get_sparse_core_info()
    nc, ns = info.num_cores, info.num_subcores
    assert r == 2 * nc
    ep = e // ns
    assert ep * ns == e and ep % _CHUNK == 0
    nch = ep // _CHUNK
    rel_per_core = r // nc
    slab = rel_per_core * n
    zslab = ns * _ZCH
    acc_rows = ((slab + ns + zslab - 1) // zslab) * zslab
    zrows = acc_rows // ns
    nfc = slab // _FCH
    assert nfc * _FCH == slab

    out_type = [jax.ShapeDtypeStruct((r * n, emb), _F32)] if with_counts \
        else jax.ShapeDtypeStruct((r * n, emb), _F32)
    scratch = [
        pltpu.VMEM((_CHUNK,), jnp.int32),
        pltpu.VMEM((_CHUNK,), jnp.int32),
        pltpu.VMEM((_CHUNK,), jnp.int32),
        pltpu.VMEM((_CHUNK,), jnp.int32),
        pltpu.VMEM((_CHUNK,), jnp.int32),
        pltpu.VMEM((_CHUNK, emb), _F32),
        pltpu.VMEM((_ZCH, emb), _F32),
        pltpu.VMEM((_FCH, emb), _F32),
        pltpu.VMEM_SHARED((acc_rows, emb), _F32),
        pltpu.SemaphoreType.DMA,
    ]
    if with_counts:
        out_type.append(jax.ShapeDtypeStruct((r * n, _L), _F32))
        scratch += [
            pltpu.VMEM((_CHUNK, _L), _F32),
            pltpu.VMEM((_ZCH, _L), _F32),
            pltpu.VMEM((_FCH, _L), _F32),
            pltpu.VMEM_SHARED((acc_rows, _L), _F32),
        ]

    def body(h_hbm, et_hbm, src_hbm, dst_hbm, *rest):
        if with_counts:
            (acc_hbm, cnt_hbm, etb, srcb, dstb, gidx, sidx, rows, zb, fb,
             acc_sh, gsem, onesb, zcb, fcb, cnt_sh) = rest
        else:
            (acc_hbm, etb, srcb, dstb, gidx, sidx, rows, zb, fb,
             acc_sh, gsem) = rest
        c = lax.axis_index("c")
        s = lax.axis_index("s")

        zero = jnp.zeros((_L,), _F32)
        one = jnp.ones((_L,), _F32)

        @pl.loop(0, _ZCH)
        def _fill(i):
            for k in range(emb // _L):
                zb[i, k * _L:(k + 1) * _L] = zero
            if with_counts:
                zcb[i, 0:_L] = zero
                onesb[i, 0:_L] = one

        @pl.loop(0, zrows // _ZCH)
        def _zero(i):
            pltpu.sync_copy(zb, acc_sh.at[pl.ds(s * zrows + i * _ZCH, _ZCH)])
            if with_counts:
                pltpu.sync_copy(zcb, cnt_sh.at[pl.ds(s * zrows + i * _ZCH, _ZCH)])

        plsc.subcore_barrier()

        base = s * ep
        dump = slab + s

        @pl.loop(0, nch)
        def _edges(j):
            off = base + j * _CHUNK
            pltpu.sync_copy(et_hbm.at[pl.ds(off, _CHUNK)], etb)
            pltpu.sync_copy(src_hbm.at[pl.ds(off, _CHUNK)], srcb)
            pltpu.sync_copy(dst_hbm.at[pl.ds(off, _CHUNK)], dstb)
            for k in range(_CHUNK // _L):
                sl = slice(k * _L, (k + 1) * _L)
                t = etb[sl]
                gidx[sl] = t * n + srcb[sl]
                lt = t - rel_per_core * c
                ok = (lt >= 0) & (lt < rel_per_core)
                sidx[sl] = jnp.where(ok, lt * n + dstb[sl], dump)
            pltpu.async_copy(h_hbm.at[gidx], rows, gsem).wait()
            pltpu.sync_copy(rows, acc_sh.at[sidx], add=True)
            if with_counts:
                pltpu.sync_copy(onesb, cnt_sh.at[sidx], add=True)

        plsc.subcore_barrier()

        @pl.loop(0, (nfc + ns - 1) // ns)
        def _flush(i):
            ch = i * ns + s

            @pl.when(ch < nfc)
            def _():
                row0 = ch * _FCH
                pltpu.sync_copy(acc_sh.at[pl.ds(row0, _FCH)], fb)
                pltpu.sync_copy(fb, acc_hbm.at[pl.ds(c * slab + row0, _FCH)])
                if with_counts:
                    pltpu.sync_copy(cnt_sh.at[pl.ds(row0, _FCH)], fcb)
                    pltpu.sync_copy(fcb, cnt_hbm.at[pl.ds(c * slab + row0, _FCH)])

    return pl.kernel(body, out_type=out_type, mesh=_mesh(),
                     scratch_types=scratch,
                     compiler_params=pltpu.CompilerParams(
                         use_tc_tiling_on_sc=False))


@functools.lru_cache(maxsize=None)
def _make_row_gather(n, b, emb):
    info = plsc.get_sparse_core_info()
    nc, ns = info.num_cores, info.num_subcores
    nw = nc * ns
    rpt = b // nw
    assert rpt * nw == b and rpt <= 128 and rpt % 8 == 0

    def body(h_hbm, idx_hbm, out_hbm, ib, rows, sem):
        c = lax.axis_index("c")
        s = lax.axis_index("s")
        base = (s * nc + c) * rpt
        pltpu.sync_copy(idx_hbm.at[pl.ds(base, rpt)], ib)
        pltpu.async_copy(h_hbm.at[ib], rows, sem).wait()
        pltpu.sync_copy(rows, out_hbm.at[pl.ds(base, rpt)])

    return pl.kernel(
        body,
        out_type=jax.ShapeDtypeStruct((b, emb), _F32),
        mesh=_mesh(),
        scratch_types=[
            pltpu.VMEM((rpt,), jnp.int32),
            pltpu.VMEM((rpt, emb), _F32),
            pltpu.SemaphoreType.DMA,
        ],
        compiler_params=pltpu.CompilerParams(use_tc_tiling_on_sc=False),
    )


def _tc_transform(x, w_rel, w_root, b, bn):
    n, fin = x.shape
    r, _, emb = w_rel.shape
    assert n % bn == 0

    def body(x_ref, wr_ref, wroot_ref, b_ref, h_ref, root_ref):
        xb = x_ref[...]
        for i in range(r):
            h_ref[i] = jnp.dot(xb, wr_ref[i], preferred_element_type=_F32)
        root_ref[...] = (jnp.dot(xb, wroot_ref[...], preferred_element_type=_F32)
                         + b_ref[...])

    return pl.pallas_call(
        body,
        grid=(n // bn,),
        in_specs=[
            pl.BlockSpec((bn, fin), lambda i: (i, 0)),
            pl.BlockSpec((r, fin, emb), lambda i: (0, 0, 0)),
            pl.BlockSpec((fin, emb), lambda i: (0, 0)),
            pl.BlockSpec((1, emb), lambda i: (0, 0)),
        ],
        out_specs=[
            pl.BlockSpec((r, bn, emb), lambda i: (0, i, 0)),
            pl.BlockSpec((bn, emb), lambda i: (i, 0)),
        ],
        out_shape=[
            jax.ShapeDtypeStruct((r, n, emb), _F32),
            jax.ShapeDtypeStruct((n, emb), _F32),
        ],
    )(x, w_rel, w_root, b)


def _tc_norm_transform(acc, cnt, root, w_rel, w_root, b, bn):
    r, n, emb = acc.shape
    emb2 = w_rel.shape[2]

    def body(acc_ref, cnt_ref, root_ref, wr_ref, wroot_ref, b_ref,
             h_ref, root2_ref):
        inv = 1.0 / jnp.maximum(cnt_ref[:, :, 0:1], 1.0)
        m = root_ref[...]
        for i in range(r):
            m = m + acc_ref[i] * inv[i]
        out1 = jnp.maximum(m, 0.0)
        for i in range(r):
            h_ref[i] = jnp.dot(out1, wr_ref[i], preferred_element_type=_F32)
        root2_ref[...] = (jnp.dot(out1, wroot_ref[...],
                                  preferred_element_type=_F32) + b_ref[...])

    return pl.pallas_call(
        body,
        grid=(n // bn,),
        in_specs=[
            pl.BlockSpec((r, bn, emb), lambda i: (0, i, 0)),
            pl.BlockSpec((r, bn, _L), lambda i: (0, i, 0)),
            pl.BlockSpec((bn, emb), lambda i: (i, 0)),
            pl.BlockSpec((r, emb, emb2), lambda i: (0, 0, 0)),
            pl.BlockSpec((emb, emb2), lambda i: (0, 0)),
            pl.BlockSpec((1, emb2), lambda i: (0, 0)),
        ],
        out_specs=[
            pl.BlockSpec((r, bn, emb2), lambda i: (0, i, 0)),
            pl.BlockSpec((bn, emb2), lambda i: (i, 0)),
        ],
        out_shape=[
            jax.ShapeDtypeStruct((r, n, emb2), _F32),
            jax.ShapeDtypeStruct((n, emb2), _F32),
        ],
    )(acc, cnt, root, w_rel, w_root, b)


def _tc_norm(acc, cnt, root, bn):
    r, n, emb = acc.shape

    def body(acc_ref, cnt_ref, root_ref, h_ref):
        inv = 1.0 / jnp.maximum(cnt_ref[:, :, 0:1], 1.0)
        m = root_ref[...]
        for i in range(r):
            m = m + acc_ref[i] * inv[i]
        h_ref[...] = m

    return pl.pallas_call(
        body,
        grid=(n // bn,),
        in_specs=[
            pl.BlockSpec((r, bn, emb), lambda i: (0, i, 0)),
            pl.BlockSpec((r, bn, _L), lambda i: (0, i, 0)),
            pl.BlockSpec((bn, emb), lambda i: (i, 0)),
        ],
        out_specs=pl.BlockSpec((bn, emb), lambda i: (i, 0)),
        out_shape=jax.ShapeDtypeStruct((n, emb), _F32),
    )(acc, cnt, root)


def _tc_heads(nest_e, food_e, w_fc_n, w_fc_f, b_fc, w_out, b_out):
    bsz, emb = nest_e.shape
    hid = w_fc_n.shape[1]
    vout = w_out.shape[1]

    def body(ne, fe, wn, wf, bf, wo, bo, o_ref):
        hidden = jnp.maximum(
            jnp.dot(ne[...], wn[...], preferred_element_type=_F32)
            + jnp.dot(fe[...], wf[...], preferred_element_type=_F32)
            + bf[...], 0.0)
        o_ref[...] = (jnp.dot(hidden, wo[...], preferred_element_type=_F32)
                      + bo[...])

    return pl.pallas_call(
        body,
        out_shape=jax.ShapeDtypeStruct((bsz, vout), _F32),
    )(nest_e, food_e, w_fc_n, w_fc_f, b_fc, w_out, b_out)


def kernel(x, edge_index, edge_type, nest_tensor, food_tensor,
           W_rel1, W_root1, b1, W_rel2, W_root2, b2,
           W_fc, b_fc, W_dir, b_dir, W_dist, b_dist):
    n, fin = x.shape
    e = edge_type.shape[0]
    r = W_rel1.shape[0]
    emb = W_rel1.shape[2]
    bsz = nest_tensor.shape[0]
    bn = 1000

    src = edge_index[0]
    dst = edge_index[1]

    h1, root1 = _tc_transform(x, W_rel1, W_root1, b1.reshape(1, -1), bn)
    acc1, cnt = _make_edge_agg(n, e, r, emb, True)(
        h1.reshape(r * n, emb), edge_type, src, dst)
    cnt3 = cnt.reshape(r, n, _L)
    h2, root2 = _tc_norm_transform(acc1.reshape(r, n, emb), cnt3, root1,
                                   W_rel2, W_root2, b2.reshape(1, -1), bn)
    acc2 = _make_edge_agg(n, e, r, emb, False)(
        h2.reshape(r * n, emb), edge_type, src, dst)
    h = _tc_norm(acc2.reshape(r, n, emb), cnt3, root2, bn)

    idx_all = jnp.concatenate([nest_tensor, food_tensor]).astype(jnp.int32)
    emb_all = _make_row_gather(n, 2 * bsz, emb)(h, idx_all)

    w_out = jnp.concatenate([W_dir, W_dist], axis=1)
    b_out = jnp.concatenate([b_dir, b_dist]).reshape(1, -1)
    out = _tc_heads(emb_all[:bsz], emb_all[bsz:], W_fc[:emb], W_fc[emb:],
                    b_fc.reshape(1, -1), w_out, b_out)
    vocab = W_dir.shape[1]
    return out[:, :vocab], out[:, vocab:]


# speedup vs baseline: 20.5978x; 3.0895x over previous
import functools

import jax
import jax.numpy as jnp
from jax import lax
from jax.experimental import pallas as pl
from jax.experimental.pallas import tpu as pltpu
from jax.experimental.pallas import tpu_sc as plsc

_F32 = jnp.float32
_L = 16
_CHUNK = 80
_ZCH = 80
_FCH = 80


def _mesh():
    return plsc.VectorSubcoreMesh(core_axis_name="c", subcore_axis_name="s")


_SB = 2000


@functools.lru_cache(maxsize=None)
def _make_edge_agg(n, e, r, emb, with_counts):
    info = plsc.get_sparse_core_info()
    nc, ns = info.num_cores, info.num_subcores
    assert r == 2 * nc
    ep = e // ns
    nsub = _SB // _CHUNK
    nstage = ep // _SB
    assert ep * ns == e and nstage * _SB == ep and nsub * _CHUNK == _SB
    rel_per_core = r // nc
    slab = rel_per_core * n
    zslab = ns * _ZCH
    acc_rows = ((slab + ns + zslab - 1) // zslab) * zslab
    zrows = acc_rows // ns
    frows = (slab // ns) & ~7
    ftail = slab - frows * ns
    assert ftail % 8 == 0
    w = 3 if with_counts else 6

    out_type = [jax.ShapeDtypeStruct((r * n, emb), _F32)] if with_counts \
        else jax.ShapeDtypeStruct((r * n, emb), _F32)
    scratch = [
        pltpu.VMEM((3, _SB), jnp.int32),
        pltpu.VMEM((nsub, _CHUNK), jnp.int32),
        pltpu.VMEM((nsub, _CHUNK), jnp.int32),
        pltpu.VMEM((w, _CHUNK, emb), _F32),
        pltpu.VMEM_SHARED((acc_rows, emb), _F32),
        pltpu.SemaphoreType.DMA,
        pltpu.SemaphoreType.DMA,
        pltpu.SemaphoreType.DMA,
    ]
    if with_counts:
        out_type.append(jax.ShapeDtypeStruct((r * n, _L), _F32))
        scratch += [
            pltpu.VMEM((_CHUNK, _L), _F32),
            pltpu.VMEM((_ZCH, _L), _F32),
            pltpu.VMEM_SHARED((acc_rows, _L), _F32),
        ]

    def body(h_hbm, et_hbm, src_hbm, dst_hbm, *rest):
        if with_counts:
            (acc_hbm, cnt_hbm, pkb, gidx, sidx, rows,
             acc_sh, gsem, ssem, fsem, onesb, zcb, cnt_sh) = rest
        else:
            (acc_hbm, pkb, gidx, sidx, rows,
             acc_sh, gsem, ssem, fsem) = rest
        c = lax.axis_index("c")
        s = lax.axis_index("s")

        zero = jnp.zeros((_L,), _F32)
        one = jnp.ones((_L,), _F32)

        @pl.loop(0, _ZCH)
        def _fill(i):
            for k in range(emb // _L):
                rows[0, i, k * _L:(k + 1) * _L] = zero
            if with_counts:
                zcb[i, 0:_L] = zero
                onesb[i, 0:_L] = one

        @pl.loop(0, zrows // _ZCH)
        def _zero(i):
            pltpu.sync_copy(rows.at[0],
                            acc_sh.at[pl.ds(s * zrows + i * _ZCH, _ZCH)])
            if with_counts:
                pltpu.sync_copy(zcb,
                                cnt_sh.at[pl.ds(s * zrows + i * _ZCH, _ZCH)])

        plsc.subcore_barrier()

        base = s * ep
        dump = slab + s

        @pl.loop(0, nstage)
        def _edges(jb):
            off = base + jb * _SB
            for i, a in enumerate((et_hbm, src_hbm, dst_hbm)):
                pltpu.sync_copy(a.at[pl.ds(off, _SB)], pkb.at[i])

            for q in range(_SB // _L):
                b, wq = divmod(q, _CHUNK // _L)
                sl = slice(q * _L, (q + 1) * _L)
                dsl = slice(wq * _L, (wq + 1) * _L)
                t = pkb[0, sl]
                gidx[b, dsl] = t * n + pkb[1, sl]
                lt = t - rel_per_core * c
                ok = (lt >= 0) & (lt < rel_per_core)
                sidx[b, dsl] = jnp.where(ok, lt * n + pkb[2, sl], dump)

            gd = [None] * nsub
            sd = [None] * nsub
            for b in range(min(w - 1, nsub)):
                gd[b] = pltpu.async_copy(h_hbm.at[gidx.at[b]],
                                         rows.at[b % w], gsem)
            for b in range(nsub):
                gd[b].wait()
                sd[b] = [pltpu.async_copy(rows.at[b % w],
                                          acc_sh.at[sidx.at[b]], ssem,
                                          add=True)]
                if with_counts:
                    sd[b].append(pltpu.async_copy(
                        onesb, cnt_sh.at[sidx.at[b]], ssem, add=True))
                f = b + w - 1
                if f < nsub:
                    if b >= 1:
                        for d in sd[b - 1]:
                            d.wait()
                        sd[b - 1] = []
                    gd[f] = pltpu.async_copy(h_hbm.at[gidx.at[f]],
                                             rows.at[f % w], gsem)
            for b in range(nsub):
                for d in sd[b] or ():
                    d.wait()

        plsc.subcore_barrier()

        fd = [pltpu.async_copy(acc_sh.at[pl.ds(s * frows, frows)],
                               acc_hbm.at[pl.ds(c * slab + s * frows, frows)],
                               fsem)]
        if with_counts:
            fd.append(pltpu.async_copy(
                cnt_sh.at[pl.ds(s * frows, frows)],
                cnt_hbm.at[pl.ds(c * slab + s * frows, frows)], fsem))
        if ftail:
            @pl.when(s == 0)
            def _():
                t0 = ns * frows
                fd2 = [pltpu.async_copy(
                    acc_sh.at[pl.ds(t0, ftail)],
                    acc_hbm.at[pl.ds(c * slab + t0, ftail)], fsem)]
                if with_counts:
                    fd2.append(pltpu.async_copy(
                        cnt_sh.at[pl.ds(t0, ftail)],
                        cnt_hbm.at[pl.ds(c * slab + t0, ftail)], fsem))
                for d in fd2:
                    d.wait()
        for d in fd:
            d.wait()

    return pl.kernel(body, out_type=out_type, mesh=_mesh(),
                     scratch_types=scratch,
                     compiler_params=pltpu.CompilerParams(
                         use_tc_tiling_on_sc=False))


@functools.lru_cache(maxsize=None)
def _make_row_gather(n, b, emb):
    info = plsc.get_sparse_core_info()
    nc, ns = info.num_cores, info.num_subcores
    nw = nc * ns
    rpt = b // nw
    assert rpt * nw == b and rpt <= 128 and rpt % 8 == 0

    def body(h_hbm, idx_hbm, out_hbm, ib, rows, sem):
        c = lax.axis_index("c")
        s = lax.axis_index("s")
        base = (s * nc + c) * rpt
        pltpu.sync_copy(idx_hbm.at[pl.ds(base, rpt)], ib)
        pltpu.async_copy(h_hbm.at[ib], rows, sem).wait()
        pltpu.sync_copy(rows, out_hbm.at[pl.ds(base, rpt)])

    return pl.kernel(
        body,
        out_type=jax.ShapeDtypeStruct((b, emb), _F32),
        mesh=_mesh(),
        scratch_types=[
            pltpu.VMEM((rpt,), jnp.int32),
            pltpu.VMEM((rpt, emb), _F32),
            pltpu.SemaphoreType.DMA,
        ],
        compiler_params=pltpu.CompilerParams(use_tc_tiling_on_sc=False),
    )


def _tc_transform(x, w_rel, w_root, b, bn):
    n, fin = x.shape
    r, _, emb = w_rel.shape
    assert n % bn == 0

    def body(x_ref, wr_ref, wroot_ref, b_ref, h_ref, root_ref):
        xb = x_ref[...]
        for i in range(r):
            h_ref[i] = jnp.dot(xb, wr_ref[i], preferred_element_type=_F32)
        root_ref[...] = (jnp.dot(xb, wroot_ref[...], preferred_element_type=_F32)
                         + b_ref[...])

    return pl.pallas_call(
        body,
        grid=(n // bn,),
        in_specs=[
            pl.BlockSpec((bn, fin), lambda i: (i, 0)),
            pl.BlockSpec((r, fin, emb), lambda i: (0, 0, 0)),
            pl.BlockSpec((fin, emb), lambda i: (0, 0)),
            pl.BlockSpec((1, emb), lambda i: (0, 0)),
        ],
        out_specs=[
            pl.BlockSpec((r, bn, emb), lambda i: (0, i, 0)),
            pl.BlockSpec((bn, emb), lambda i: (i, 0)),
        ],
        out_shape=[
            jax.ShapeDtypeStruct((r, n, emb), _F32),
            jax.ShapeDtypeStruct((n, emb), _F32),
        ],
    )(x, w_rel, w_root, b)


def _tc_norm_transform(acc, cnt, root, w_rel, w_root, b, bn):
    r, n, emb = acc.shape
    emb2 = w_rel.shape[2]

    def body(acc_ref, cnt_ref, root_ref, wr_ref, wroot_ref, b_ref,
             h_ref, root2_ref):
        inv = 1.0 / jnp.maximum(cnt_ref[:, :, 0:1], 1.0)
        m = root_ref[...]
        for i in range(r):
            m = m + acc_ref[i] * inv[i]
        out1 = jnp.maximum(m, 0.0)
        for i in range(r):
            h_ref[i] = jnp.dot(out1, wr_ref[i], preferred_element_type=_F32)
        root2_ref[...] = (jnp.dot(out1, wroot_ref[...],
                                  preferred_element_type=_F32) + b_ref[...])

    return pl.pallas_call(
        body,
        grid=(n // bn,),
        in_specs=[
            pl.BlockSpec((r, bn, emb), lambda i: (0, i, 0)),
            pl.BlockSpec((r, bn, _L), lambda i: (0, i, 0)),
            pl.BlockSpec((bn, emb), lambda i: (i, 0)),
            pl.BlockSpec((r, emb, emb2), lambda i: (0, 0, 0)),
            pl.BlockSpec((emb, emb2), lambda i: (0, 0)),
            pl.BlockSpec((1, emb2), lambda i: (0, 0)),
        ],
        out_specs=[
            pl.BlockSpec((r, bn, emb2), lambda i: (0, i, 0)),
            pl.BlockSpec((bn, emb2), lambda i: (i, 0)),
        ],
        out_shape=[
            jax.ShapeDtypeStruct((r, n, emb2), _F32),
            jax.ShapeDtypeStruct((n, emb2), _F32),
        ],
    )(acc, cnt, root, w_rel, w_root, b)


def _tc_norm(acc, cnt, root, bn):
    r, n, emb = acc.shape

    def body(acc_ref, cnt_ref, root_ref, h_ref):
        inv = 1.0 / jnp.maximum(cnt_ref[:, :, 0:1], 1.0)
        m = root_ref[...]
        for i in range(r):
            m = m + acc_ref[i] * inv[i]
        h_ref[...] = m

    return pl.pallas_call(
        body,
        grid=(n // bn,),
        in_specs=[
            pl.BlockSpec((r, bn, emb), lambda i: (0, i, 0)),
            pl.BlockSpec((r, bn, _L), lambda i: (0, i, 0)),
            pl.BlockSpec((bn, emb), lambda i: (i, 0)),
        ],
        out_specs=pl.BlockSpec((bn, emb), lambda i: (i, 0)),
        out_shape=jax.ShapeDtypeStruct((n, emb), _F32),
    )(acc, cnt, root)


def _tc_heads(nest_e, food_e, w_fc_n, w_fc_f, b_fc, w_out, b_out):
    bsz, emb = nest_e.shape
    hid = w_fc_n.shape[1]
    vout = w_out.shape[1]

    def body(ne, fe, wn, wf, bf, wo, bo, o_ref):
        hidden = jnp.maximum(
            jnp.dot(ne[...], wn[...], preferred_element_type=_F32)
            + jnp.dot(fe[...], wf[...], preferred_element_type=_F32)
            + bf[...], 0.0)
        o_ref[...] = (jnp.dot(hidden, wo[...], preferred_element_type=_F32)
                      + bo[...])

    return pl.pallas_call(
        body,
        out_shape=jax.ShapeDtypeStruct((bsz, vout), _F32),
    )(nest_e, food_e, w_fc_n, w_fc_f, b_fc, w_out, b_out)


def kernel(x, edge_index, edge_type, nest_tensor, food_tensor,
           W_rel1, W_root1, b1, W_rel2, W_root2, b2,
           W_fc, b_fc, W_dir, b_dir, W_dist, b_dist):
    n, fin = x.shape
    e = edge_type.shape[0]
    r = W_rel1.shape[0]
    emb = W_rel1.shape[2]
    bsz = nest_tensor.shape[0]
    bn = 1000

    src = edge_index[0]
    dst = edge_index[1]

    h1, root1 = _tc_transform(x, W_rel1, W_root1, b1.reshape(1, -1), bn)
    acc1, cnt = _make_edge_agg(n, e, r, emb, True)(
        h1.reshape(r * n, emb), edge_type, src, dst)
    cnt3 = cnt.reshape(r, n, _L)
    h2, root2 = _tc_norm_transform(acc1.reshape(r, n, emb), cnt3, root1,
                                   W_rel2, W_root2, b2.reshape(1, -1), bn)
    acc2 = _make_edge_agg(n, e, r, emb, False)(
        h2.reshape(r * n, emb), edge_type, src, dst)
    h = _tc_norm(acc2.reshape(r, n, emb), cnt3, root2, bn)

    idx_all = jnp.concatenate([nest_tensor, food_tensor]).astype(jnp.int32)
    emb_all = _make_row_gather(n, 2 * bsz, emb)(h, idx_all)

    w_out = jnp.concatenate([W_dir, W_dist], axis=1)
    b_out = jnp.concatenate([b_dir, b_dist]).reshape(1, -1)
    out = _tc_heads(emb_all[:bsz], emb_all[bsz:], W_fc[:emb], W_fc[emb:],
                    b_fc.reshape(1, -1), w_out, b_out)
    vocab = W_dir.shape[1]
    return out[:, :vocab], out[:, vocab:]


# speedup vs baseline: 25.0539x; 1.2163x over previous
import functools

import jax
import jax.numpy as jnp
from jax import lax
from jax.experimental import pallas as pl
from jax.experimental.pallas import tpu as pltpu
from jax.experimental.pallas import tpu_sc as plsc

_F32 = jnp.float32
_L = 16
_CHUNK = 80
_ZCH = 80
_FCH = 80


def _mesh():
    return plsc.VectorSubcoreMesh(core_axis_name="c", subcore_axis_name="s")


_SB = 2000


@functools.lru_cache(maxsize=None)
def _make_counts(n, e, r):
    info = plsc.get_sparse_core_info()
    nc, ns = info.num_cores, info.num_subcores
    ep = e // ns
    nsub = _SB // _CHUNK
    nstage = ep // _SB
    assert ep * ns == e and nstage * _SB == ep
    rel_per_core = r // nc
    slab = rel_per_core * n
    zslab = ns * _ZCH
    acc_rows = ((slab + ns + zslab - 1) // zslab) * zslab
    zrows = acc_rows // ns
    frows = (slab // ns) & ~7
    ftail = slab - frows * ns

    def body(et_hbm, dst_hbm, cnt_hbm, pk0, pk1, sidx, onesb, zcb,
             cnt_sh, ssem, fsem):
        c = lax.axis_index("c")
        s = lax.axis_index("s")
        zero = jnp.zeros((_L,), _F32)
        one = jnp.ones((_L,), _F32)

        @pl.loop(0, _ZCH)
        def _fill(i):
            zcb[i, 0:_L] = zero

        @pl.loop(0, _CHUNK)
        def _fill2(i):
            onesb[i, 0:_L] = one

        @pl.loop(0, zrows // _ZCH)
        def _zero(i):
            pltpu.sync_copy(zcb, cnt_sh.at[pl.ds(s * zrows + i * _ZCH, _ZCH)])

        plsc.subcore_barrier()

        base = s * ep
        dump = slab + s
        pks = (pk0, pk1)

        for i, a in enumerate((et_hbm, dst_hbm)):
            pltpu.sync_copy(a.at[pl.ds(base, _SB)], pk0.at[i])

        @pl.loop(0, nstage, step=2)
        def _blocks(jb):
            for p in range(2):
                blk = jb + p
                pkb = pks[p]

                @pl.when(blk > 0)
                def _():
                    for i, a in enumerate((et_hbm, dst_hbm)):
                        pltpu.make_async_copy(
                            a.at[pl.ds(base + blk * _SB, _SB)],
                            pkb.at[i], fsem).wait()

                @pl.when(blk + 1 < nstage)
                def _():
                    for i, a in enumerate((et_hbm, dst_hbm)):
                        pltpu.async_copy(
                            a.at[pl.ds(base + (blk + 1) * _SB, _SB)],
                            pks[1 - p].at[i], fsem)

                for q in range(_SB // _L):
                    b, wq = divmod(q, _CHUNK // _L)
                    sl = slice(q * _L, (q + 1) * _L)
                    dsl = slice(wq * _L, (wq + 1) * _L)
                    t = pkb[0, sl]
                    lt = t - rel_per_core * c
                    ok = (lt >= 0) & (lt < rel_per_core)
                    sidx[b, dsl] = jnp.where(ok, lt * n + pkb[1, sl], dump)

                sd = [pltpu.async_copy(onesb, cnt_sh.at[sidx.at[b]],
                                       ssem, add=True)
                      for b in range(nsub)]
                for d in sd:
                    d.wait()

        plsc.subcore_barrier()

        fd = [pltpu.async_copy(cnt_sh.at[pl.ds(s * frows, frows)],
                               cnt_hbm.at[pl.ds(c * slab + s * frows, frows)],
                               fsem)]
        if ftail:
            @pl.when(s == 0)
            def _():
                t0 = ns * frows
                pltpu.async_copy(cnt_sh.at[pl.ds(t0, ftail)],
                                 cnt_hbm.at[pl.ds(c * slab + t0, ftail)],
                                 fsem).wait()
        for d in fd:
            d.wait()

    return pl.kernel(
        body,
        out_type=jax.ShapeDtypeStruct((r * n, _L), _F32),
        mesh=_mesh(),
        scratch_types=[
            pltpu.VMEM((2, _SB), jnp.int32),
            pltpu.VMEM((2, _SB), jnp.int32),
            pltpu.VMEM((_SB // _CHUNK, _CHUNK), jnp.int32),
            pltpu.VMEM((_CHUNK, _L), _F32),
            pltpu.VMEM((_ZCH, _L), _F32),
            pltpu.VMEM_SHARED((acc_rows, _L), _F32),
            pltpu.SemaphoreType.DMA,
            pltpu.SemaphoreType.DMA,
        ],
        compiler_params=pltpu.CompilerParams(use_tc_tiling_on_sc=False))


@functools.lru_cache(maxsize=None)
def _make_edge_agg(n, e, r, emb):
    info = plsc.get_sparse_core_info()
    nc, ns = info.num_cores, info.num_subcores
    assert r == 2 * nc
    ep = e // ns
    nsub = _SB // _CHUNK
    nstage = ep // _SB
    assert ep * ns == e and nstage * _SB == ep and nstage % 2 == 0
    rel_per_core = r // nc
    slab = rel_per_core * n
    zslab = ns * _ZCH
    acc_rows = ((slab + ns + zslab - 1) // zslab) * zslab
    zrows = acc_rows // ns
    frows = (slab // ns) & ~7
    ftail = slab - frows * ns
    w = 5

    def body(h_hbm, et_hbm, src_hbm, dst_hbm, acc_hbm,
             pk0, pk1, gidx, sidx, rows, acc_sh, gsem, ssem, fsem):
        c = lax.axis_index("c")
        s = lax.axis_index("s")
        zero = jnp.zeros((_L,), _F32)

        @pl.loop(0, _ZCH)
        def _fill(i):
            for k in range(emb // _L):
                rows[0, i, k * _L:(k + 1) * _L] = zero

        @pl.loop(0, zrows // _ZCH)
        def _zero(i):
            pltpu.sync_copy(rows.at[0],
                            acc_sh.at[pl.ds(s * zrows + i * _ZCH, _ZCH)])

        plsc.subcore_barrier()

        base = s * ep
        dump = slab + s
        pks = (pk0, pk1)

        for i, a in enumerate((et_hbm, src_hbm, dst_hbm)):
            pltpu.sync_copy(a.at[pl.ds(base, _SB)], pk0.at[i])

        @pl.loop(0, nstage, step=2)
        def _edges(jb):
            for p in range(2):
                blk = jb + p
                pkb = pks[p]

                @pl.when(blk > 0)
                def _():
                    for i, a in enumerate((et_hbm, src_hbm, dst_hbm)):
                        pltpu.make_async_copy(
                            a.at[pl.ds(base + blk * _SB, _SB)],
                            pkb.at[i], fsem).wait()

                @pl.when(blk + 1 < nstage)
                def _():
                    for i, a in enumerate((et_hbm, src_hbm, dst_hbm)):
                        pltpu.async_copy(
                            a.at[pl.ds(base + (blk + 1) * _SB, _SB)],
                            pks[1 - p].at[i], fsem)

                for q in range(_SB // _L):
                    b, wq = divmod(q, _CHUNK // _L)
                    sl = slice(q * _L, (q + 1) * _L)
                    dsl = slice(wq * _L, (wq + 1) * _L)
                    t = pkb[0, sl]
                    gidx[b, dsl] = t * n + pkb[1, sl]
                    lt = t - rel_per_core * c
                    ok = (lt >= 0) & (lt < rel_per_core)
                    sidx[b, dsl] = jnp.where(ok, lt * n + pkb[2, sl], dump)

                gd = [None] * nsub
                sd = [None] * nsub
                for b in range(min(w - 1, nsub)):
                    gd[b] = pltpu.async_copy(h_hbm.at[gidx.at[b]],
                                             rows.at[b % w], gsem)
                for b in range(nsub):
                    gd[b].wait()
                    sd[b] = pltpu.async_copy(rows.at[b % w],
                                             acc_sh.at[sidx.at[b]], ssem,
                                             add=True)
                    f = b + w - 1
                    if f < nsub:
                        if b >= 1:
                            sd[b - 1].wait()
                            sd[b - 1] = None
                        gd[f] = pltpu.async_copy(h_hbm.at[gidx.at[f]],
                                                 rows.at[f % w], gsem)
                for d in sd:
                    if d is not None:
                        d.wait()

        plsc.subcore_barrier()

        fd = [pltpu.async_copy(acc_sh.at[pl.ds(s * frows, frows)],
                               acc_hbm.at[pl.ds(c * slab + s * frows, frows)],
                               fsem)]
        if ftail:
            @pl.when(s == 0)
            def _():
                t0 = ns * frows
                pltpu.async_copy(acc_sh.at[pl.ds(t0, ftail)],
                                 acc_hbm.at[pl.ds(c * slab + t0, ftail)],
                                 fsem).wait()
        for d in fd:
            d.wait()

    return pl.kernel(
        body,
        out_type=jax.ShapeDtypeStruct((r * n, emb), _F32),
        mesh=_mesh(),
        scratch_types=[
            pltpu.VMEM((3, _SB), jnp.int32),
            pltpu.VMEM((3, _SB), jnp.int32),
            pltpu.VMEM((_SB // _CHUNK, _CHUNK), jnp.int32),
            pltpu.VMEM((_SB // _CHUNK, _CHUNK), jnp.int32),
            pltpu.VMEM((w, _CHUNK, emb), _F32),
            pltpu.VMEM_SHARED((acc_rows, emb), _F32),
            pltpu.SemaphoreType.DMA,
            pltpu.SemaphoreType.DMA,
            pltpu.SemaphoreType.DMA,
        ],
        compiler_params=pltpu.CompilerParams(use_tc_tiling_on_sc=False))


@functools.lru_cache(maxsize=None)
def _make_final_gather(n, b, r, emb):
    info = plsc.get_sparse_core_info()
    nc, ns = info.num_cores, info.num_subcores
    nw = nc * ns
    rpt = b // nw
    assert rpt * nw == b and rpt <= 128 and rpt % 8 == 0

    def body(root_hbm, acc_hbm, cnt_hbm, idx_hbm, parts_hbm, cparts_hbm,
             ib, idxr, rows, crows, sem):
        c = lax.axis_index("c")
        sidx = lax.axis_index("s")
        base = (sidx * nc + c) * rpt
        pltpu.sync_copy(idx_hbm.at[pl.ds(base, rpt)], ib)
        pltpu.async_copy(root_hbm.at[ib], rows, sem).wait()
        pltpu.sync_copy(rows, parts_hbm.at[0, pl.ds(base, rpt)])
        for rel in range(r):
            for q in range(rpt // _L):
                sl = slice(q * _L, (q + 1) * _L)
                idxr[sl] = ib[sl] + rel * n
            pltpu.async_copy(acc_hbm.at[idxr], rows, sem).wait()
            pltpu.sync_copy(rows, parts_hbm.at[1 + rel, pl.ds(base, rpt)])
            pltpu.async_copy(cnt_hbm.at[idxr], crows, sem).wait()
            pltpu.sync_copy(crows, cparts_hbm.at[rel, pl.ds(base, rpt)])

    return pl.kernel(
        body,
        out_type=[
            jax.ShapeDtypeStruct((1 + r, b, emb), _F32),
            jax.ShapeDtypeStruct((r, b, _L), _F32),
        ],
        mesh=_mesh(),
        scratch_types=[
            pltpu.VMEM((rpt,), jnp.int32),
            pltpu.VMEM((rpt,), jnp.int32),
            pltpu.VMEM((rpt, emb), _F32),
            pltpu.VMEM((rpt, _L), _F32),
            pltpu.SemaphoreType.DMA,
        ],
        compiler_params=pltpu.CompilerParams(use_tc_tiling_on_sc=False))


def _tc_transform(x, w_rel, w_root, b, bn):
    n, fin = x.shape
    r, _, emb = w_rel.shape
    assert n % bn == 0

    def body(x_ref, wr_ref, wroot_ref, b_ref, h_ref, root_ref):
        xb = x_ref[...]
        for i in range(r):
            h_ref[i] = jnp.dot(xb, wr_ref[i], preferred_element_type=_F32)
        root_ref[...] = (jnp.dot(xb, wroot_ref[...], preferred_element_type=_F32)
                         + b_ref[...])

    return pl.pallas_call(
        body,
        grid=(n // bn,),
        in_specs=[
            pl.BlockSpec((bn, fin), lambda i: (i, 0)),
            pl.BlockSpec((r, fin, emb), lambda i: (0, 0, 0)),
            pl.BlockSpec((fin, emb), lambda i: (0, 0)),
            pl.BlockSpec((1, emb), lambda i: (0, 0)),
        ],
        out_specs=[
            pl.BlockSpec((r, bn, emb), lambda i: (0, i, 0)),
            pl.BlockSpec((bn, emb), lambda i: (i, 0)),
        ],
        out_shape=[
            jax.ShapeDtypeStruct((r, n, emb), _F32),
            jax.ShapeDtypeStruct((n, emb), _F32),
        ],
    )(x, w_rel, w_root, b)


def _tc_norm_transform(acc, cnt, root, w_rel, w_root, b, bn):
    r, n, emb = acc.shape
    emb2 = w_rel.shape[2]

    def body(acc_ref, cnt_ref, root_ref, wr_ref, wroot_ref, b_ref,
             h_ref, root2_ref):
        inv = 1.0 / jnp.maximum(cnt_ref[:, :, 0:1], 1.0)
        m = root_ref[...]
        for i in range(r):
            m = m + acc_ref[i] * inv[i]
        out1 = jnp.maximum(m, 0.0)
        for i in range(r):
            h_ref[i] = jnp.dot(out1, wr_ref[i], preferred_element_type=_F32)
        root2_ref[...] = (jnp.dot(out1, wroot_ref[...],
                                  preferred_element_type=_F32) + b_ref[...])

    return pl.pallas_call(
        body,
        grid=(n // bn,),
        in_specs=[
            pl.BlockSpec((r, bn, emb), lambda i: (0, i, 0)),
            pl.BlockSpec((r, bn, _L), lambda i: (0, i, 0)),
            pl.BlockSpec((bn, emb), lambda i: (i, 0)),
            pl.BlockSpec((r, emb, emb2), lambda i: (0, 0, 0)),
            pl.BlockSpec((emb, emb2), lambda i: (0, 0)),
            pl.BlockSpec((1, emb2), lambda i: (0, 0)),
        ],
        out_specs=[
            pl.BlockSpec((r, bn, emb2), lambda i: (0, i, 0)),
            pl.BlockSpec((bn, emb2), lambda i: (i, 0)),
        ],
        out_shape=[
            jax.ShapeDtypeStruct((r, n, emb2), _F32),
            jax.ShapeDtypeStruct((n, emb2), _F32),
        ],
    )(acc, cnt, root, w_rel, w_root, b)


def _tc_norm(acc, cnt, root, bn):
    r, n, emb = acc.shape

    def body(acc_ref, cnt_ref, root_ref, h_ref):
        inv = 1.0 / jnp.maximum(cnt_ref[:, :, 0:1], 1.0)
        m = root_ref[...]
        for i in range(r):
            m = m + acc_ref[i] * inv[i]
        h_ref[...] = m

    return pl.pallas_call(
        body,
        grid=(n // bn,),
        in_specs=[
            pl.BlockSpec((r, bn, emb), lambda i: (0, i, 0)),
            pl.BlockSpec((r, bn, _L), lambda i: (0, i, 0)),
            pl.BlockSpec((bn, emb), lambda i: (i, 0)),
        ],
        out_specs=pl.BlockSpec((bn, emb), lambda i: (i, 0)),
        out_shape=jax.ShapeDtypeStruct((n, emb), _F32),
    )(acc, cnt, root)


def _tc_heads(parts, cparts, w_fc_n, w_fc_f, b_fc, w_out, b_out):
    r1, b2, emb = parts.shape
    r = r1 - 1
    bsz = b2 // 2
    hid = w_fc_n.shape[1]
    vout = w_out.shape[1]

    def body(p_ref, c_ref, wn, wf, bf, wo, bo, o_ref):
        h = p_ref[0]
        for i in range(r):
            inv = 1.0 / jnp.maximum(c_ref[i, :, 0:1], 1.0)
            h = h + p_ref[1 + i] * inv
        ne = h[:bsz]
        fe = h[bsz:]
        hidden = jnp.maximum(
            jnp.dot(ne, wn[...], preferred_element_type=_F32)
            + jnp.dot(fe, wf[...], preferred_element_type=_F32)
            + bf[...], 0.0)
        o_ref[...] = (jnp.dot(hidden, wo[...], preferred_element_type=_F32)
                      + bo[...])

    return pl.pallas_call(
        body,
        out_shape=jax.ShapeDtypeStruct((bsz, vout), _F32),
    )(parts, cparts, w_fc_n, w_fc_f, b_fc, w_out, b_out)


def kernel(x, edge_index, edge_type, nest_tensor, food_tensor,
           W_rel1, W_root1, b1, W_rel2, W_root2, b2,
           W_fc, b_fc, W_dir, b_dir, W_dist, b_dist):
    n, fin = x.shape
    e = edge_type.shape[0]
    r = W_rel1.shape[0]
    emb = W_rel1.shape[2]
    bsz = nest_tensor.shape[0]
    bn = 1000

    src = edge_index[0]
    dst = edge_index[1]

    cnt = _make_counts(n, e, r)(edge_type, dst)
    cnt3 = cnt.reshape(r, n, _L)
    et_dep = edge_type + (cnt[0, 0] * 0.0).astype(jnp.int32)
    h1, root1 = _tc_transform(x, W_rel1, W_root1, b1.reshape(1, -1), bn)
    acc1 = _make_edge_agg(n, e, r, emb)(
        h1.reshape(r * n, emb), et_dep, src, dst)
    h2, root2 = _tc_norm_transform(acc1.reshape(r, n, emb), cnt3, root1,
                                   W_rel2, W_root2, b2.reshape(1, -1), bn)
    acc2 = _make_edge_agg(n, e, r, emb)(
        h2.reshape(r * n, emb), edge_type, src, dst)

    idx_all = jnp.concatenate([nest_tensor, food_tensor]).astype(jnp.int32)
    parts, cparts = _make_final_gather(n, 2 * bsz, r, emb)(
        root2, acc2, cnt, idx_all)

    w_out = jnp.concatenate([W_dir, W_dist], axis=1)
    b_out = jnp.concatenate([b_dir, b_dist]).reshape(1, -1)
    out = _tc_heads(parts, cparts, W_fc[:emb], W_fc[emb:],
                    b_fc.reshape(1, -1), w_out, b_out)
    vocab = W_dir.shape[1]
    return out[:, :vocab], out[:, vocab:]


# speedup vs baseline: 25.0541x; 1.0000x over previous
import functools

import jax
import jax.numpy as jnp
from jax import lax
from jax.experimental import pallas as pl
from jax.experimental.pallas import tpu as pltpu
from jax.experimental.pallas import tpu_sc as plsc

_F32 = jnp.float32
_L = 16
_CHUNK = 80
_ZCH = 80
_FCH = 80


def _mesh():
    return plsc.VectorSubcoreMesh(core_axis_name="c", subcore_axis_name="s")


_SB = 2000


@functools.lru_cache(maxsize=None)
def _make_counts(n, e, r):
    info = plsc.get_sparse_core_info()
    nc, ns = info.num_cores, info.num_subcores
    ep = e // ns
    nsub = _SB // _CHUNK
    nstage = ep // _SB
    assert ep * ns == e and nstage * _SB == ep
    rel_per_core = r // nc
    slab = rel_per_core * n
    zslab = ns * _ZCH
    acc_rows = ((slab + ns + zslab - 1) // zslab) * zslab
    zrows = acc_rows // ns
    frows = (slab // ns) & ~7
    ftail = slab - frows * ns

    def body(et_hbm, dst_hbm, cnt_hbm, pk0, pk1, sidx, onesb, zcb,
             cnt_sh, ssem, fsem):
        c = lax.axis_index("c")
        s = lax.axis_index("s")
        zero = jnp.zeros((_L,), _F32)
        one = jnp.ones((_L,), _F32)

        @pl.loop(0, _ZCH)
        def _fill(i):
            zcb[i, 0:_L] = zero

        @pl.loop(0, _CHUNK)
        def _fill2(i):
            onesb[i, 0:_L] = one

        @pl.loop(0, zrows // _ZCH)
        def _zero(i):
            pltpu.sync_copy(zcb, cnt_sh.at[pl.ds(s * zrows + i * _ZCH, _ZCH)])

        plsc.subcore_barrier()

        base = s * ep
        dump = slab + s
        pks = (pk0, pk1)

        for i, a in enumerate((et_hbm, dst_hbm)):
            pltpu.sync_copy(a.at[pl.ds(base, _SB)], pk0.at[i])

        @pl.loop(0, nstage, step=2)
        def _blocks(jb):
            for p in range(2):
                blk = jb + p
                pkb = pks[p]

                @pl.when(blk > 0)
                def _():
                    for i, a in enumerate((et_hbm, dst_hbm)):
                        pltpu.make_async_copy(
                            a.at[pl.ds(base + blk * _SB, _SB)],
                            pkb.at[i], fsem).wait()

                @pl.when(blk + 1 < nstage)
                def _():
                    for i, a in enumerate((et_hbm, dst_hbm)):
                        pltpu.async_copy(
                            a.at[pl.ds(base + (blk + 1) * _SB, _SB)],
                            pks[1 - p].at[i], fsem)

                for q in range(_SB // _L):
                    b, wq = divmod(q, _CHUNK // _L)
                    sl = slice(q * _L, (q + 1) * _L)
                    dsl = slice(wq * _L, (wq + 1) * _L)
                    t = pkb[0, sl]
                    lt = t - rel_per_core * c
                    ok = (lt >= 0) & (lt < rel_per_core)
                    sidx[b, dsl] = jnp.where(ok, lt * n + pkb[1, sl], dump)

                sd = [pltpu.async_copy(onesb, cnt_sh.at[sidx.at[b]],
                                       ssem, add=True)
                      for b in range(nsub)]
                for d in sd:
                    d.wait()

        plsc.subcore_barrier()

        fd = [pltpu.async_copy(cnt_sh.at[pl.ds(s * frows, frows)],
                               cnt_hbm.at[pl.ds(c * slab + s * frows, frows)],
                               fsem)]
        if ftail:
            @pl.when(s == 0)
            def _():
                t0 = ns * frows
                pltpu.async_copy(cnt_sh.at[pl.ds(t0, ftail)],
                                 cnt_hbm.at[pl.ds(c * slab + t0, ftail)],
                                 fsem).wait()
        for d in fd:
            d.wait()

    return pl.kernel(
        body,
        out_type=jax.ShapeDtypeStruct((r * n, _L), _F32),
        mesh=_mesh(),
        scratch_types=[
            pltpu.VMEM((2, _SB), jnp.int32),
            pltpu.VMEM((2, _SB), jnp.int32),
            pltpu.VMEM((_SB // _CHUNK, _CHUNK), jnp.int32),
            pltpu.VMEM((_CHUNK, _L), _F32),
            pltpu.VMEM((_ZCH, _L), _F32),
            pltpu.VMEM_SHARED((acc_rows, _L), _F32),
            pltpu.SemaphoreType.DMA,
            pltpu.SemaphoreType.DMA,
        ],
        compiler_params=pltpu.CompilerParams(use_tc_tiling_on_sc=False))


@functools.lru_cache(maxsize=None)
def _make_edge_agg(n, e, r, emb):
    info = plsc.get_sparse_core_info()
    nc, ns = info.num_cores, info.num_subcores
    assert r == 2 * nc
    ep = e // ns
    nsub = _SB // _CHUNK
    nstage = ep // _SB
    assert ep * ns == e and nstage * _SB == ep and nstage % 2 == 0
    rel_per_core = r // nc
    slab = rel_per_core * n
    zslab = ns * _ZCH
    acc_rows = ((slab + ns + zslab - 1) // zslab) * zslab
    zrows = acc_rows // ns
    frows = (slab // ns) & ~7
    ftail = slab - frows * ns
    w = 5

    def body(h_hbm, et_hbm, src_hbm, dst_hbm, acc_hbm,
             pk0, pk1, gidx, sidx, rows, acc_sh, gsem, ssem, fsem):
        c = lax.axis_index("c")
        s = lax.axis_index("s")
        zero = jnp.zeros((_L,), _F32)

        @pl.loop(0, _ZCH)
        def _fill(i):
            for k in range(emb // _L):
                rows[0, i, k * _L:(k + 1) * _L] = zero

        @pl.loop(0, zrows // _ZCH)
        def _zero(i):
            pltpu.sync_copy(rows.at[0],
                            acc_sh.at[pl.ds(s * zrows + i * _ZCH, _ZCH)])

        plsc.subcore_barrier()

        base = s * ep
        dump = slab + s
        pks = (pk0, pk1)

        for i, a in enumerate((et_hbm, src_hbm, dst_hbm)):
            pltpu.sync_copy(a.at[pl.ds(base, _SB)], pk0.at[i])

        @pl.loop(0, nstage, step=2)
        def _edges(jb):
            for p in range(2):
                blk = jb + p
                pkb = pks[p]

                @pl.when(blk > 0)
                def _():
                    for i, a in enumerate((et_hbm, src_hbm, dst_hbm)):
                        pltpu.make_async_copy(
                            a.at[pl.ds(base + blk * _SB, _SB)],
                            pkb.at[i], fsem).wait()

                @pl.when(blk + 1 < nstage)
                def _():
                    for i, a in enumerate((et_hbm, src_hbm, dst_hbm)):
                        pltpu.async_copy(
                            a.at[pl.ds(base + (blk + 1) * _SB, _SB)],
                            pks[1 - p].at[i], fsem)

                for q in range(_SB // _L):
                    b, wq = divmod(q, _CHUNK // _L)
                    sl = slice(q * _L, (q + 1) * _L)
                    dsl = slice(wq * _L, (wq + 1) * _L)
                    t = pkb[0, sl]
                    gidx[b, dsl] = t * n + pkb[1, sl]
                    lt = t - rel_per_core * c
                    ok = (lt >= 0) & (lt < rel_per_core)
                    sidx[b, dsl] = jnp.where(ok, lt * n + pkb[2, sl], dump)

                gd = [None] * nsub
                sd = [None] * nsub
                for b in range(min(w - 1, nsub)):
                    gd[b] = pltpu.async_copy(h_hbm.at[gidx.at[b]],
                                             rows.at[b % w], gsem)
                for b in range(nsub):
                    gd[b].wait()
                    sd[b] = pltpu.async_copy(rows.at[b % w],
                                             acc_sh.at[sidx.at[b]], ssem,
                                             add=True)
                    f = b + w - 1
                    if f < nsub:
                        if b >= 1:
                            sd[b - 1].wait()
                            sd[b - 1] = None
                        gd[f] = pltpu.async_copy(h_hbm.at[gidx.at[f]],
                                                 rows.at[f % w], gsem)
                for d in sd:
                    if d is not None:
                        d.wait()

        plsc.subcore_barrier()

        fd = [pltpu.async_copy(acc_sh.at[pl.ds(s * frows, frows)],
                               acc_hbm.at[pl.ds(c * slab + s * frows, frows)],
                               fsem)]
        if ftail:
            @pl.when(s == 0)
            def _():
                t0 = ns * frows
                pltpu.async_copy(acc_sh.at[pl.ds(t0, ftail)],
                                 acc_hbm.at[pl.ds(c * slab + t0, ftail)],
                                 fsem).wait()
        for d in fd:
            d.wait()

    return pl.kernel(
        body,
        out_type=jax.ShapeDtypeStruct((r * n, emb), _F32),
        mesh=_mesh(),
        scratch_types=[
            pltpu.VMEM((3, _SB), jnp.int32),
            pltpu.VMEM((3, _SB), jnp.int32),
            pltpu.VMEM((_SB // _CHUNK, _CHUNK), jnp.int32),
            pltpu.VMEM((_SB // _CHUNK, _CHUNK), jnp.int32),
            pltpu.VMEM((w, _CHUNK, emb), _F32),
            pltpu.VMEM_SHARED((acc_rows, emb), _F32),
            pltpu.SemaphoreType.DMA,
            pltpu.SemaphoreType.DMA,
            pltpu.SemaphoreType.DMA,
        ],
        compiler_params=pltpu.CompilerParams(use_tc_tiling_on_sc=False))


@functools.lru_cache(maxsize=None)
def _make_final_gather(n, b, r, emb):
    info = plsc.get_sparse_core_info()
    nc, ns = info.num_cores, info.num_subcores
    nw = nc * ns
    rpt = b // nw
    assert rpt * nw == b and rpt <= 128 and rpt % 8 == 0

    def body(root_hbm, acc_hbm, cnt_hbm, idx_hbm, parts_hbm, cparts_hbm,
             ib, idxr, rows, crows, sem):
        c = lax.axis_index("c")
        sidx = lax.axis_index("s")
        base = (sidx * nc + c) * rpt
        pltpu.sync_copy(idx_hbm.at[pl.ds(base, rpt)], ib)
        pltpu.async_copy(root_hbm.at[ib], rows, sem).wait()
        pltpu.sync_copy(rows, parts_hbm.at[0, pl.ds(base, rpt)])
        for rel in range(r):
            for q in range(rpt // _L):
                sl = slice(q * _L, (q + 1) * _L)
                idxr[sl] = ib[sl] + rel * n
            pltpu.async_copy(acc_hbm.at[idxr], rows, sem).wait()
            pltpu.sync_copy(rows, parts_hbm.at[1 + rel, pl.ds(base, rpt)])
            pltpu.async_copy(cnt_hbm.at[idxr], crows, sem).wait()
            pltpu.sync_copy(crows, cparts_hbm.at[rel, pl.ds(base, rpt)])

    return pl.kernel(
        body,
        out_type=[
            jax.ShapeDtypeStruct((1 + r, b, emb), _F32),
            jax.ShapeDtypeStruct((r, b, _L), _F32),
        ],
        mesh=_mesh(),
        scratch_types=[
            pltpu.VMEM((rpt,), jnp.int32),
            pltpu.VMEM((rpt,), jnp.int32),
            pltpu.VMEM((rpt, emb), _F32),
            pltpu.VMEM((rpt, _L), _F32),
            pltpu.SemaphoreType.DMA,
        ],
        compiler_params=pltpu.CompilerParams(use_tc_tiling_on_sc=False))


def _tc_transform(x, w_rel, w_root, b, bn):
    n, fin = x.shape
    r, _, emb = w_rel.shape
    assert n % bn == 0

    def body(x_ref, wr_ref, wroot_ref, b_ref, h_ref, root_ref):
        xb = x_ref[...]
        for i in range(r):
            h_ref[i] = jnp.dot(xb, wr_ref[i], preferred_element_type=_F32)
        root_ref[...] = (jnp.dot(xb, wroot_ref[...], preferred_element_type=_F32)
                         + b_ref[...])

    return pl.pallas_call(
        body,
        grid=(n // bn,),
        in_specs=[
            pl.BlockSpec((bn, fin), lambda i: (i, 0)),
            pl.BlockSpec((r, fin, emb), lambda i: (0, 0, 0)),
            pl.BlockSpec((fin, emb), lambda i: (0, 0)),
            pl.BlockSpec((1, emb), lambda i: (0, 0)),
        ],
        out_specs=[
            pl.BlockSpec((r, bn, emb), lambda i: (0, i, 0)),
            pl.BlockSpec((bn, emb), lambda i: (i, 0)),
        ],
        out_shape=[
            jax.ShapeDtypeStruct((r, n, emb), _F32),
            jax.ShapeDtypeStruct((n, emb), _F32),
        ],
    )(x, w_rel, w_root, b)


def _tc_norm_transform(acc, cnt, root, w_rel, w_root, b, bn):
    r, n, emb = acc.shape
    emb2 = w_rel.shape[2]

    def body(acc_ref, cnt_ref, root_ref, wr_ref, wroot_ref, b_ref,
             h_ref, root2_ref):
        inv = 1.0 / jnp.maximum(cnt_ref[:, :, 0:1], 1.0)
        m = root_ref[...]
        for i in range(r):
            m = m + acc_ref[i] * inv[i]
        out1 = jnp.maximum(m, 0.0)
        for i in range(r):
            h_ref[i] = jnp.dot(out1, wr_ref[i], preferred_element_type=_F32)
        root2_ref[...] = (jnp.dot(out1, wroot_ref[...],
                                  preferred_element_type=_F32) + b_ref[...])

    return pl.pallas_call(
        body,
        grid=(n // bn,),
        in_specs=[
            pl.BlockSpec((r, bn, emb), lambda i: (0, i, 0)),
            pl.BlockSpec((r, bn, _L), lambda i: (0, i, 0)),
            pl.BlockSpec((bn, emb), lambda i: (i, 0)),
            pl.BlockSpec((r, emb, emb2), lambda i: (0, 0, 0)),
            pl.BlockSpec((emb, emb2), lambda i: (0, 0)),
            pl.BlockSpec((1, emb2), lambda i: (0, 0)),
        ],
        out_specs=[
            pl.BlockSpec((r, bn, emb2), lambda i: (0, i, 0)),
            pl.BlockSpec((bn, emb2), lambda i: (i, 0)),
        ],
        out_shape=[
            jax.ShapeDtypeStruct((r, n, emb2), _F32),
            jax.ShapeDtypeStruct((n, emb2), _F32),
        ],
    )(acc, cnt, root, w_rel, w_root, b)


def _tc_heads(parts, cparts, w_fc_n, w_fc_f, b_fc, w_out, b_out):
    r1, b2, emb = parts.shape
    r = r1 - 1
    bsz = b2 // 2
    hid = w_fc_n.shape[1]
    vout = w_out.shape[1]

    def body(p_ref, c_ref, wn, wf, bf, wo, bo, o_ref):
        h = p_ref[0]
        for i in range(r):
            inv = 1.0 / jnp.maximum(c_ref[i, :, 0:1], 1.0)
            h = h + p_ref[1 + i] * inv
        ne = h[:bsz]
        fe = h[bsz:]
        hidden = jnp.maximum(
            jnp.dot(ne, wn[...], preferred_element_type=_F32)
            + jnp.dot(fe, wf[...], preferred_element_type=_F32)
            + bf[...], 0.0)
        o_ref[...] = (jnp.dot(hidden, wo[...], preferred_element_type=_F32)
                      + bo[...])

    return pl.pallas_call(
        body,
        out_shape=jax.ShapeDtypeStruct((bsz, vout), _F32),
    )(parts, cparts, w_fc_n, w_fc_f, b_fc, w_out, b_out)


def kernel(x, edge_index, edge_type, nest_tensor, food_tensor,
           W_rel1, W_root1, b1, W_rel2, W_root2, b2,
           W_fc, b_fc, W_dir, b_dir, W_dist, b_dist):
    n, fin = x.shape
    e = edge_type.shape[0]
    r = W_rel1.shape[0]
    emb = W_rel1.shape[2]
    bsz = nest_tensor.shape[0]
    bn = 1000

    src = edge_index[0]
    dst = edge_index[1]

    cnt = _make_counts(n, e, r)(edge_type, dst)
    cnt3 = cnt.reshape(r, n, _L)
    et_dep = edge_type + (cnt[0, 0] * 0.0).astype(jnp.int32)
    h1, root1 = _tc_transform(x, W_rel1, W_root1, b1.reshape(1, -1), bn)
    acc1 = _make_edge_agg(n, e, r, emb)(
        h1.reshape(r * n, emb), et_dep, src, dst)
    h2, root2 = _tc_norm_transform(acc1.reshape(r, n, emb), cnt3, root1,
                                   W_rel2, W_root2, b2.reshape(1, -1), bn)
    acc2 = _make_edge_agg(n, e, r, emb)(
        h2.reshape(r * n, emb), edge_type, src, dst)

    idx_all = jnp.concatenate([nest_tensor, food_tensor]).astype(jnp.int32)
    parts, cparts = _make_final_gather(n, 2 * bsz, r, emb)(
        root2, acc2, cnt, idx_all)

    w_out = jnp.concatenate([W_dir, W_dist], axis=1)
    b_out = jnp.concatenate([b_dir, b_dist]).reshape(1, -1)
    out = _tc_heads(parts, cparts, W_fc[:emb], W_fc[emb:],
                    b_fc.reshape(1, -1), w_out, b_out)
    vocab = W_dir.shape[1]
    return out[:, :vocab], out[:, vocab:]
